# baseline (speedup 1.0000x reference)
import functools
import jax
import jax.numpy as jnp
from jax import lax
from jax.experimental import pallas as pl
from jax.experimental.pallas import tpu as pltpu
from jax.experimental.pallas import tpu_sc as plsc

S = 512
H = 128
R = 16
K = 128
T = 1024


def _sc_factor_body(rel_hbm, grep_hbm, colbuf, rowbuf):
    wid = lax.axis_index("s") * 2 + lax.axis_index("c")
    pltpu.sync_copy(rel_hbm.at[wid], colbuf)
    for c in range(2):
        acc = lax.fori_loop(
            0, H,
            lambda h, a: a + colbuf[pl.ds(h * 32 + c * 16, 16)],
            jnp.zeros((16,), jnp.float32),
        )
        g16 = 1.0 + (0.1 / H) * acc
        for r in range(16):
            idx = jnp.full((16,), r, jnp.int32)
            splat = g16.at[idx].get(mode="promise_in_bounds")
            for cc in range(8):
                rowbuf[pl.ds((c * 16 + r) * H + cc * 16, 16)] = splat
    pltpu.sync_copy(rowbuf, grep_hbm.at[pl.ds(wid * 32 * H, 32 * H)])


def _sc_factor(rel_tiles):
    mesh = plsc.VectorSubcoreMesh(core_axis_name="c", subcore_axis_name="s")
    return pl.kernel(
        _sc_factor_body,
        mesh=mesh,
        out_type=jax.ShapeDtypeStruct((T * H,), jnp.float32),
        scratch_types=[
            pltpu.VMEM((H * 32,), jnp.float32),
            pltpu.VMEM((32 * H,), jnp.float32),
        ],
    )(rel_tiles)


def _tca_body(pose_ref, pos_ref, relrev_ref, out_ref, emb_ref, grep_ref):
    p = pl.program_id(0)

    @pl.when(p == 0)
    def _init():
        emb_ref[...] = pose_ref[0] + pos_ref[...]
        m = jnp.mean(relrev_ref[...], axis=1, keepdims=True)
        grep_ref[...] = jnp.broadcast_to(1.0 + 0.1 * m, (T, H))

    i0 = p * R
    emb = emb_ref[...]
    for r in range(R):
        start = (S - 1) - (i0 + r)
        out_ref[0, r] = emb * grep_ref[pl.ds(start, S), :]


def _tcb_body(pose_ref, pos_ref, grep_ref, outa_ref, out_ref, emb_ref):
    del outa_ref
    p = pl.program_id(0)

    @pl.when(p == 0)
    def _init():
        emb_ref[...] = pose_ref[0] + pos_ref[...]

    i0 = K + p * R
    emb = emb_ref[...]
    for r in range(R):
        start = (S - 1) - (i0 + r)
        out_ref[0, r] = emb * grep_ref[pl.ds(start, S), :]


def kernel(pose_features, pos_emb_table, rel_table):
    relrev = jnp.concatenate(
        [jnp.flip(rel_table, axis=0), jnp.zeros((1, H), jnp.float32)], axis=0
    )
    rel_tiles = jnp.transpose(relrev.T.reshape(H, 32, 32), (1, 0, 2)).reshape(32, H * 32)

    grep = _sc_factor(rel_tiles).reshape(T, H)

    out_shape = jax.ShapeDtypeStruct((1, S, S, H), jnp.float32)
    outa = pl.pallas_call(
        _tca_body,
        grid=(K // R,),
        in_specs=[
            pl.BlockSpec((1, S, H), lambda p: (0, 0, 0)),
            pl.BlockSpec((S, H), lambda p: (0, 0)),
            pl.BlockSpec((T, H), lambda p: (0, 0)),
        ],
        out_specs=pl.BlockSpec((1, R, S, H), lambda p: (0, p, 0, 0)),
        out_shape=out_shape,
        scratch_shapes=[
            pltpu.VMEM((S, H), jnp.float32),
            pltpu.VMEM((T, H), jnp.float32),
        ],
    )(pose_features, pos_emb_table, relrev)

    out = pl.pallas_call(
        _tcb_body,
        grid=((S - K) // R,),
        in_specs=[
            pl.BlockSpec((1, S, H), lambda p: (0, 0, 0)),
            pl.BlockSpec((S, H), lambda p: (0, 0)),
            pl.BlockSpec((T, H), lambda p: (0, 0)),
            pl.BlockSpec(memory_space=pl.ANY),
        ],
        out_specs=pl.BlockSpec((1, R, S, H), lambda p: (0, K // R + p, 0, 0)),
        out_shape=out_shape,
        scratch_shapes=[pltpu.VMEM((S, H), jnp.float32)],
        input_output_aliases={3: 0},
    )(pose_features, pos_emb_table, grep, outa)
    return out


# speedup vs baseline: 1.3376x; 1.3376x over previous
import functools
import jax
import jax.numpy as jnp
from jax import lax
from jax.experimental import pallas as pl
from jax.experimental.pallas import tpu as pltpu
from jax.experimental.pallas import tpu_sc as plsc

S = 512
H = 128
R = 16
K = 128
T = 1024


def _sc_factor_body(rel_hbm, grep_hbm, colbuf, rowbuf):
    wid = lax.axis_index("s") * 2 + lax.axis_index("c")
    pltpu.sync_copy(rel_hbm.at[wid], colbuf)
    for c in range(2):
        acc = lax.fori_loop(
            0, H,
            lambda h, a: a + colbuf[pl.ds(h * 32 + c * 16, 16)],
            jnp.zeros((16,), jnp.float32),
        )
        g16 = 1.0 + (0.1 / H) * acc
        for r in range(16):
            idx = jnp.full((16,), r, jnp.int32)
            splat = g16.at[idx].get(mode="promise_in_bounds")
            for cc in range(8):
                rowbuf[pl.ds((c * 16 + r) * H + cc * 16, 16)] = splat
    pltpu.sync_copy(rowbuf, grep_hbm.at[pl.ds(wid * 32 * H, 32 * H)])


def _sc_factor(rel_tiles):
    mesh = plsc.VectorSubcoreMesh(core_axis_name="c", subcore_axis_name="s")
    return pl.kernel(
        _sc_factor_body,
        mesh=mesh,
        out_type=jax.ShapeDtypeStruct((T * H,), jnp.float32),
        scratch_types=[
            pltpu.VMEM((H * 32,), jnp.float32),
            pltpu.VMEM((32 * H,), jnp.float32),
        ],
    )(rel_tiles)


def _tca_body(pose_ref, pos_ref, relrev_ref, out_ref, emb_ref, grep_ref):
    p = pl.program_id(0)

    @pl.when(p == 0)
    def _init():
        emb_ref[...] = pose_ref[0] + pos_ref[...]
        m = jnp.mean(relrev_ref[...], axis=1, keepdims=True)
        grep_ref[...] = jnp.broadcast_to(1.0 + 0.1 * m, (T, H))

    i0 = p * R
    emb = emb_ref[...]
    for r in range(R):
        start = (S - 1) - (i0 + r)
        out_ref[0, r] = emb * grep_ref[pl.ds(start, S), :]


def _tcb_body(pose_ref, pos_ref, relrev_ref, outa_ref, out_ref, emb_ref, grep_ref):
    del outa_ref
    p = pl.program_id(0)

    @pl.when(p == 0)
    def _init():
        emb_ref[...] = pose_ref[0] + pos_ref[...]
        m = jnp.mean(relrev_ref[...], axis=1, keepdims=True)
        grep_ref[...] = jnp.broadcast_to(1.0 + 0.1 * m, (T, H))

    i0 = K + p * R
    emb = emb_ref[...]
    for r in range(R):
        start = (S - 1) - (i0 + r)
        out_ref[0, r] = emb * grep_ref[pl.ds(start, S), :]


def kernel(pose_features, pos_emb_table, rel_table):
    relrev = jnp.concatenate(
        [jnp.flip(rel_table, axis=0), jnp.zeros((1, H), jnp.float32)], axis=0
    )
    rel_tiles = jnp.transpose(relrev.T.reshape(H, 32, 32), (1, 0, 2)).reshape(32, H * 32)


    out_shape = jax.ShapeDtypeStruct((1, S, S, H), jnp.float32)
    outa = pl.pallas_call(
        _tca_body,
        grid=(K // R,),
        in_specs=[
            pl.BlockSpec((1, S, H), lambda p: (0, 0, 0)),
            pl.BlockSpec((S, H), lambda p: (0, 0)),
            pl.BlockSpec((T, H), lambda p: (0, 0)),
        ],
        out_specs=pl.BlockSpec((1, R, S, H), lambda p: (0, p, 0, 0)),
        out_shape=out_shape,
        scratch_shapes=[
            pltpu.VMEM((S, H), jnp.float32),
            pltpu.VMEM((T, H), jnp.float32),
        ],
    )(pose_features, pos_emb_table, relrev)

    out = pl.pallas_call(
        _tcb_body,
        grid=((S - K) // R,),
        in_specs=[
            pl.BlockSpec((1, S, H), lambda p: (0, 0, 0)),
            pl.BlockSpec((S, H), lambda p: (0, 0)),
            pl.BlockSpec((T, H), lambda p: (0, 0)),
            pl.BlockSpec(memory_space=pl.ANY),
        ],
        out_specs=pl.BlockSpec((1, R, S, H), lambda p: (0, K // R + p, 0, 0)),
        out_shape=out_shape,
        scratch_shapes=[
            pltpu.VMEM((S, H), jnp.float32),
            pltpu.VMEM((T, H), jnp.float32),
        ],
        input_output_aliases={3: 0},
    )(pose_features, pos_emb_table, relrev, outa)
    return out


# speedup vs baseline: 1.3975x; 1.0448x over previous
import functools
import jax
import jax.numpy as jnp
from jax.experimental import pallas as pl
from jax.experimental.pallas import tpu as pltpu

S = 512
H = 128
R = 16
T = 1024


def _body(pose_ref, pos_ref, rel_ref, out_ref, emb_ref, grep_ref):
    p = pl.program_id(0)

    @pl.when(p == 0)
    def _init():
        emb_ref[...] = pose_ref[0] + pos_ref[...]
        m = jnp.mean(rel_ref[...], axis=1, keepdims=True)
        grep_ref[...] = jnp.broadcast_to(1.0 + 0.1 * m, (T, H))

    i0 = p * R
    emb = emb_ref[...]
    for r in range(R):
        start = (S - 1) - (i0 + r)
        out_ref[0, r] = emb * grep_ref[pl.ds(start, S), :]


def kernel(pose_features, pos_emb_table, rel_table):
    grid = S // R
    out = pl.pallas_call(
        _body,
        grid=(grid,),
        in_specs=[
            pl.BlockSpec((1, S, H), lambda p: (0, 0, 0)),
            pl.BlockSpec((S, H), lambda p: (0, 0)),
            pl.BlockSpec((T, H), lambda p: (0, 0)),
        ],
        out_specs=pl.BlockSpec((1, R, S, H), lambda p: (0, p, 0, 0)),
        out_shape=jax.ShapeDtypeStruct((1, S, S, H), jnp.float32),
        scratch_shapes=[
            pltpu.VMEM((S, H), jnp.float32),
            pltpu.VMEM((T, H), jnp.float32),
        ],
    )(pose_features, pos_emb_table, rel_table[::-1])
    return out


# speedup vs baseline: 1.4351x; 1.0269x over previous
import functools
import jax
import jax.numpy as jnp
from jax.experimental import pallas as pl
from jax.experimental.pallas import tpu as pltpu

S = 512
H = 128
R = 16
T = 1024


def _body(pose_ref, pos_ref, rel_ref, out_ref, emb_ref, grep_ref):
    p = pl.program_id(0)

    @pl.when(p == 0)
    def _init():
        emb_ref[...] = pose_ref[0] + pos_ref[...]
        m = jnp.mean(rel_ref[...], axis=1, keepdims=True)
        grep_ref[...] = jnp.broadcast_to(1.0 + 0.1 * m, (T, H))

    i0 = p * R
    emb = emb_ref[...]
    for r in range(R):
        start = S - (i0 + r)
        out_ref[0, r] = emb * grep_ref[pl.ds(start, S), :]


def kernel(pose_features, pos_emb_table, rel_table):
    grid = S // R
    out = pl.pallas_call(
        _body,
        grid=(grid,),
        in_specs=[
            pl.BlockSpec((1, S, H), lambda p: (0, 0, 0)),
            pl.BlockSpec((S, H), lambda p: (0, 0)),
            pl.BlockSpec((T, H), lambda p: (0, 0)),
        ],
        out_specs=pl.BlockSpec((1, R, S, H), lambda p: (0, p, 0, 0)),
        out_shape=jax.ShapeDtypeStruct((1, S, S, H), jnp.float32),
        scratch_shapes=[
            pltpu.VMEM((S, H), jnp.float32),
            pltpu.VMEM((T, H), jnp.float32),
        ],
    )(pose_features, pos_emb_table,
      jnp.flip(jnp.pad(rel_table, ((0, 1), (0, 0))), axis=0))
    return out


# speedup vs baseline: 1.5460x; 1.0772x over previous
import functools
import jax
import jax.numpy as jnp
from jax.experimental import pallas as pl
from jax.experimental.pallas import tpu as pltpu

S = 512
H = 128
R = 16
T = 1024


def _body(pose_ref, pos_ref, rel_ref, out_ref, emb_ref, grep_ref):
    p = pl.program_id(0)

    @pl.when(p == 0)
    def _init():
        emb_ref[...] = pose_ref[0] + pos_ref[...]
        m = jnp.mean(rel_ref[...], axis=1, keepdims=True)
        g = 1.0 + 0.1 * m
        s = jax.lax.broadcasted_iota(jnp.int32, (T, 1), 0)
        for k in (4, 2, 1):
            g = jnp.where((s % (2 * k)) < k,
                          pltpu.roll(g, T - k, 0), pltpu.roll(g, k, 0))
        for b in range(T // 8):
            blk = jax.lax.slice(g, (8 * b, 0), (8 * b + 8, 1))
            grep_ref[pl.ds(8 * (T // 8 - 1 - b), 8), :] = jnp.broadcast_to(
                blk, (8, H))

    i0 = p * R
    emb = emb_ref[...]
    for r in range(R):
        start = S - (i0 + r)
        out_ref[0, r] = emb * grep_ref[pl.ds(start, S), :]


def kernel(pose_features, pos_emb_table, rel_table):
    grid = S // R
    out = pl.pallas_call(
        _body,
        grid=(grid,),
        in_specs=[
            pl.BlockSpec((1, S, H), lambda p: (0, 0, 0)),
            pl.BlockSpec((S, H), lambda p: (0, 0)),
            pl.BlockSpec((T, H), lambda p: (0, 0)),
        ],
        out_specs=pl.BlockSpec((1, R, S, H), lambda p: (0, p, 0, 0)),
        out_shape=jax.ShapeDtypeStruct((1, S, S, H), jnp.float32),
        scratch_shapes=[
            pltpu.VMEM((S, H), jnp.float32),
            pltpu.VMEM((T, H), jnp.float32),
        ],
    )(pose_features, pos_emb_table, rel_table)
    return out
